# baseline (speedup 1.0000x reference)
import jax
import jax.numpy as jnp
from jax import lax
from jax.experimental import pallas as pl
from jax.experimental.pallas import tpu as pltpu

B, S, D = 4, 8192, 1024
N = B * S
V = 8192
RT = 512
NT = N


def _tc_body(idx_ref, tok_ref, pos_ref, out_ref):
  i = pl.program_id(0)

  def row(r, carry):
    j = idx_ref[i * RT + r]
    out_ref[r] = tok_ref[r] + pos_ref[j]
    return carry

  lax.fori_loop(0, RT, row, 0, unroll=8)


_grid_spec = pltpu.PrefetchScalarGridSpec(
    num_scalar_prefetch=1,
    grid=(NT // RT,),
    in_specs=[
        pl.BlockSpec((RT, 8, 128), lambda i, idx: (i, 0, 0)),
        pl.BlockSpec((V, 8, 128), lambda i, idx: (0, 0, 0)),
    ],
    out_specs=pl.BlockSpec((RT, 8, 128), lambda i, idx: (i, 0, 0)),
)

_tc_call = pl.pallas_call(
    _tc_body,
    grid_spec=_grid_spec,
    out_shape=jax.ShapeDtypeStruct((NT, 8, 128), jnp.float32),
)


@jax.jit
def kernel(tokens, pos_indices, pos_enc):
  tok3 = tokens.reshape(NT, 8, 128)
  idx = pos_indices.reshape(N).astype(jnp.int32)
  pos3 = pos_enc.reshape(V, 8, 128)
  out = _tc_call(idx, tok3, pos3)
  return out.reshape(B, S, D)
